# baseline (speedup 1.0000x reference)
import functools

import jax
import jax.numpy as jnp
from jax import lax
from jax.experimental import pallas as pl
from jax.experimental.pallas import tpu as pltpu
from jax.experimental.pallas import tpu_sc as plsc

_NUM_EMB = 1000
_EMB_DIM = 128
_BATCH = 16384
_MARGIN = 1.0
_EPS = 1e-6
_NPAD = 1024
_NHALF = _NPAD // 2
_NWORDS = _NHALF * _NPAD

_NC = 1
_NS = 16
_NW = _NC * _NS
_BPW = _BATCH // _NW
_CH = 128
_NCH = _BPW // _CH
_LANES = 16


def _sqrt16(x):
    xi = lax.bitcast_convert_type(x, jnp.int32)
    r = lax.bitcast_convert_type(
        jnp.int32(0x5F3759DF) - lax.shift_right_arithmetic(xi, 1), jnp.float32)
    for _ in range(3):
        r = r * (1.5 - 0.5 * x * r * r)
    return x * r


_GRID = 4
_HROWS = _NHALF // _GRID


def _bf16_bits(d2):
    bits = lax.bitcast_convert_type(d2, jnp.int32)
    return lax.shift_right_logical(
        bits + 0x7FFF + (lax.shift_right_logical(bits, 16) & 1), 16)


def _dist_matrix_body(t_ref, a_ref, p_ref, n_ref,
                      d_ref, iap_ref, ian_ref, sha_ref, lhs_ref, rhs_ref):
    g = pl.program_id(0)

    @pl.when(g == 0)
    def _prologue():
        av = a_ref[...]
        arow = (av & (_NHALF - 1)) * _NPAD
        iap_ref[...] = arow + p_ref[...]
        ian_ref[...] = arow + n_ref[...]
        sha_ref[...] = (1 - (av >> 9)) << 4

        t = jnp.concatenate(
            [t_ref[...], jnp.zeros((_NPAD - _NUM_EMB, _EMB_DIM), jnp.float32)],
            axis=0)
        sq = t * t
        n2c = jnp.sum(sq, axis=1, keepdims=True)
        rsc = jnp.sum(t, axis=1, keepdims=True)
        diag_val = _EMB_DIM * _EPS * _EPS
        u = n2c + (2.0 * _EPS) * rsc
        v = (n2c - (2.0 * _EPS) * rsc) + diag_val
        one = jnp.ones((_NPAD, 1), jnp.float32)
        lhs_ref[...] = jnp.concatenate([-2.0 * t, u, one], axis=1)
        rhs_ref[...] = jnp.concatenate([t, one, v], axis=1)

    dn = (((1,), (1,)), ((), ()))
    rhs = rhs_ref[...]
    lo = lax.dot_general(lhs_ref[pl.ds(g * _HROWS, _HROWS), :], rhs, dn)
    hi = lax.dot_general(
        lhs_ref[pl.ds(_NHALF + g * _HROWS, _HROWS), :], rhs, dn)
    w = _bf16_bits(lo) | lax.shift_left(_bf16_bits(hi), 16)
    d_ref[...] = w.reshape(_HROWS * 8, 128)


def _sc_triplet_body(dpacked, iap_hbm, ian_hbm, sha_hbm, out_hbm,
                     iap, ian, sha, wap, wan, accv, sem1, sem2):
    wid = lax.axis_index("s") * _NC + lax.axis_index("c")
    base = wid * _BPW
    cp1 = pltpu.async_copy(iap_hbm.at[pl.ds(base, _BPW)], iap, sem1)
    cp2 = pltpu.async_copy(ian_hbm.at[pl.ds(base, _BPW)], ian, sem2)
    cp3 = pltpu.async_copy(sha_hbm.at[pl.ds(base, _BPW)], sha, sem1)
    cp1.wait()
    cp2.wait()
    copies = []
    for c in range(_NCH):
        sl = pl.ds(c * _CH, _CH)
        copies.append(pltpu.async_copy(dpacked.at[iap.at[sl]], wap.at[sl], sem1))
        copies.append(pltpu.async_copy(dpacked.at[ian.at[sl]], wan.at[sl], sem2))
    cp3.wait()

    mask = jnp.full((_LANES,), jnp.int32(-65536))

    def unpack(wv, sh):
        return lax.bitcast_convert_type(lax.shift_left(wv, sh) & mask,
                                        jnp.float32)

    def step(v, acc):
        sl = pl.ds(v * _LANES, _LANES)
        sh = sha[sl]
        d_ap = _sqrt16(jnp.maximum(unpack(wap[sl], sh), 0.0))
        d_an = _sqrt16(jnp.maximum(unpack(wan[sl], sh), 0.0))
        return acc + jnp.maximum(d_ap - d_an + _MARGIN, 0.0)

    for cp in copies:
        cp.wait()
    accv[...] = lax.fori_loop(0, _BPW // _LANES,
                              step, jnp.zeros((_LANES,), jnp.float32))
    pltpu.sync_copy(accv, out_hbm.at[wid])


_sc_triplet = functools.partial(
    pl.kernel,
    out_type=jax.ShapeDtypeStruct((_NW, _LANES), jnp.float32),
    mesh=plsc.VectorSubcoreMesh(core_axis_name="c", subcore_axis_name="s",
                                num_cores=_NC),
    scratch_types=[
        pltpu.VMEM((_BPW,), jnp.int32),
        pltpu.VMEM((_BPW,), jnp.int32),
        pltpu.VMEM((_BPW,), jnp.int32),
        pltpu.VMEM((_BPW,), jnp.int32),
        pltpu.VMEM((_BPW,), jnp.int32),
        pltpu.VMEM((_LANES,), jnp.float32),
        pltpu.SemaphoreType.DMA,
        pltpu.SemaphoreType.DMA,
    ],
)(_sc_triplet_body)


def kernel(a, p, n, emb_table):
    full_b = pl.BlockSpec((_BATCH,), lambda g: (0,))
    dist, iap, ian, sha = pl.pallas_call(
        _dist_matrix_body,
        grid=(_GRID,),
        in_specs=[
            pl.BlockSpec((_NUM_EMB, _EMB_DIM), lambda g: (0, 0)),
            full_b, full_b, full_b,
        ],
        out_specs=(
            pl.BlockSpec((_HROWS * 8, 128), lambda g: (g, 0)),
            full_b, full_b, full_b,
        ),
        out_shape=(
            jax.ShapeDtypeStruct((_NHALF * 8, 128), jnp.int32),
            jax.ShapeDtypeStruct((_BATCH,), jnp.int32),
            jax.ShapeDtypeStruct((_BATCH,), jnp.int32),
            jax.ShapeDtypeStruct((_BATCH,), jnp.int32),
        ),
        scratch_shapes=[
            pltpu.VMEM((_NPAD, _EMB_DIM + 2), jnp.float32),
            pltpu.VMEM((_NPAD, _EMB_DIM + 2), jnp.float32),
        ],
    )(emb_table, a, p, n)
    partials = _sc_triplet(dist.reshape(_NWORDS), iap, ian, sha)
    return jnp.sum(partials) / _BATCH


# speedup vs baseline: 1.0189x; 1.0189x over previous
import functools

import jax
import jax.numpy as jnp
from jax import lax
from jax.experimental import pallas as pl
from jax.experimental.pallas import tpu as pltpu
from jax.experimental.pallas import tpu_sc as plsc

_NUM_EMB = 1000
_EMB_DIM = 128
_BATCH = 16384
_MARGIN = 1.0
_EPS = 1e-6
_NPAD = 1024
_NHALF = _NPAD // 2
_NWORDS = _NHALF * _NPAD

_NC = 1
_NS = 16
_NW = _NC * _NS
_BPW = _BATCH // _NW
_CH = 128
_NCH = _BPW // _CH
_LANES = 16


def _sqrt16(x):
    xi = lax.bitcast_convert_type(x, jnp.int32)
    r = lax.bitcast_convert_type(
        jnp.int32(0x5F3759DF) - lax.shift_right_arithmetic(xi, 1), jnp.float32)
    for _ in range(3):
        r = r * (1.5 - 0.5 * x * r * r)
    return x * r


def _dist_matrix_body(t_ref, a_ref, p_ref, n_ref,
                      d_ref, iap_ref, ian_ref, sha_ref):
    av = a_ref[...]
    arow = (av & (_NHALF - 1)) * _NPAD
    iap_ref[...] = arow + p_ref[...]
    ian_ref[...] = arow + n_ref[...]
    sha_ref[...] = (1 - (av >> 9)) << 4

    t = jnp.concatenate(
        [t_ref[...], jnp.zeros((_NPAD - _NUM_EMB, _EMB_DIM), jnp.float32)], axis=0)
    sq = t * t
    n2c = jnp.sum(sq, axis=1, keepdims=True)
    rsc = jnp.sum(t, axis=1, keepdims=True)
    diag_val = _EMB_DIM * _EPS * _EPS
    u = n2c + (2.0 * _EPS) * rsc
    v = (n2c - (2.0 * _EPS) * rsc) + diag_val
    one = jnp.ones((_NPAD, 1), jnp.float32)
    lhs = jnp.concatenate([-2.0 * t, u, one], axis=1)
    rhs = jnp.concatenate([t, one, v], axis=1)
    dn = (((1,), (1,)), ((), ()))
    d2 = lax.dot_general(lhs, rhs, dn)
    bits = lax.bitcast_convert_type(d2, jnp.int32)
    r16 = lax.shift_right_logical(
        bits + 0x7FFF + (lax.shift_right_logical(bits, 16) & 1), 16)
    w = r16[:_NHALF] | lax.shift_left(r16[_NHALF:], 16)
    d_ref[...] = w.reshape(_NHALF * 8, 128)


def _sc_triplet_body(dpacked, iap_hbm, ian_hbm, sha_hbm, out_hbm,
                     iap, ian, sha, wap, wan, accv, sem1, sem2):
    wid = lax.axis_index("s") * _NC + lax.axis_index("c")
    base = wid * _BPW
    cp1 = pltpu.async_copy(iap_hbm.at[pl.ds(base, _BPW)], iap, sem1)
    cp2 = pltpu.async_copy(ian_hbm.at[pl.ds(base, _BPW)], ian, sem2)
    cp3 = pltpu.async_copy(sha_hbm.at[pl.ds(base, _BPW)], sha, sem1)
    cp1.wait()
    cp2.wait()
    copies = []
    for c in range(_NCH):
        sl = pl.ds(c * _CH, _CH)
        copies.append(pltpu.async_copy(dpacked.at[iap.at[sl]], wap.at[sl], sem1))
        copies.append(pltpu.async_copy(dpacked.at[ian.at[sl]], wan.at[sl], sem2))
    cp3.wait()

    mask = jnp.full((_LANES,), jnp.int32(-65536))

    def unpack(wv, sh):
        return lax.bitcast_convert_type(lax.shift_left(wv, sh) & mask,
                                        jnp.float32)

    def step(v, acc):
        sl = pl.ds(v * _LANES, _LANES)
        sh = sha[sl]
        d_ap = _sqrt16(jnp.maximum(unpack(wap[sl], sh), 0.0))
        d_an = _sqrt16(jnp.maximum(unpack(wan[sl], sh), 0.0))
        return acc + jnp.maximum(d_ap - d_an + _MARGIN, 0.0)

    acc = jnp.zeros((_LANES,), jnp.float32)
    grp = _CH // _LANES
    for c in range(_NCH):
        copies[2 * c].wait()
        copies[2 * c + 1].wait()
        acc = lax.fori_loop(c * grp, (c + 1) * grp, step, acc)
    accv[...] = acc
    pltpu.sync_copy(accv, out_hbm.at[wid])


_sc_triplet = functools.partial(
    pl.kernel,
    out_type=jax.ShapeDtypeStruct((_NW, _LANES), jnp.float32),
    mesh=plsc.VectorSubcoreMesh(core_axis_name="c", subcore_axis_name="s",
                                num_cores=_NC),
    scratch_types=[
        pltpu.VMEM((_BPW,), jnp.int32),
        pltpu.VMEM((_BPW,), jnp.int32),
        pltpu.VMEM((_BPW,), jnp.int32),
        pltpu.VMEM((_BPW,), jnp.int32),
        pltpu.VMEM((_BPW,), jnp.int32),
        pltpu.VMEM((_LANES,), jnp.float32),
        pltpu.SemaphoreType.DMA,
        pltpu.SemaphoreType.DMA,
    ],
)(_sc_triplet_body)


def kernel(a, p, n, emb_table):
    dist, iap, ian, sha = pl.pallas_call(
        _dist_matrix_body,
        out_shape=(
            jax.ShapeDtypeStruct((_NHALF * 8, 128), jnp.int32),
            jax.ShapeDtypeStruct((_BATCH,), jnp.int32),
            jax.ShapeDtypeStruct((_BATCH,), jnp.int32),
            jax.ShapeDtypeStruct((_BATCH,), jnp.int32),
        ),
    )(emb_table, a, p, n)
    partials = _sc_triplet(dist.reshape(_NWORDS), iap, ian, sha)
    return jnp.sum(partials) / _BATCH


# speedup vs baseline: 1.0437x; 1.0243x over previous
import functools

import jax
import jax.numpy as jnp
from jax import lax
from jax.experimental import pallas as pl
from jax.experimental.pallas import tpu as pltpu
from jax.experimental.pallas import tpu_sc as plsc

_NUM_EMB = 1000
_EMB_DIM = 128
_BATCH = 16384
_MARGIN = 1.0
_EPS = 1e-6
_NPAD = 1024
_NHALF = _NPAD // 2
_NWORDS = _NHALF * _NPAD

_NC = 1
_NS = 16
_NW = _NC * _NS
_BPW = _BATCH // _NW
_CH = 128
_NCH = _BPW // _CH
_LANES = 16


def _sqrt16(x):
    xi = lax.bitcast_convert_type(x, jnp.int32)
    r = lax.bitcast_convert_type(
        jnp.int32(0x5F3759DF) - lax.shift_right_arithmetic(xi, 1), jnp.float32)
    for _ in range(3):
        r = r * (1.5 - 0.5 * x * r * r)
    return x * r


def _dist_matrix_body(t_ref, a_ref, p_ref, n_ref,
                      d_ref, iap_ref, ian_ref, sha_ref):
    av = a_ref[...]
    arow = (av & (_NHALF - 1)) * _NPAD
    iap_ref[...] = arow + p_ref[...]
    ian_ref[...] = arow + n_ref[...]
    sha_ref[...] = (1 - (av >> 9)) << 4

    t = jnp.concatenate(
        [t_ref[...], jnp.zeros((_NPAD - _NUM_EMB, _EMB_DIM), jnp.float32)], axis=0)
    sq = t * t
    n2c = jnp.sum(sq, axis=1, keepdims=True)
    rsc = jnp.sum(t, axis=1, keepdims=True)
    diag_val = _EMB_DIM * _EPS * _EPS
    u = n2c + (2.0 * _EPS) * rsc
    v = (n2c - (2.0 * _EPS) * rsc) + diag_val
    one = jnp.ones((_NPAD, 1), jnp.float32)
    lhs = jnp.concatenate([-2.0 * t, u, one], axis=1)
    rhs = jnp.concatenate([t, one, v], axis=1)
    dn = (((1,), (1,)), ((), ()))
    d2 = lax.dot_general(lhs, rhs, dn)
    bits = lax.bitcast_convert_type(d2, jnp.int32)
    r16 = lax.shift_right_logical(
        bits + 0x7FFF + (lax.shift_right_logical(bits, 16) & 1), 16)
    w = r16[:_NHALF] | lax.shift_left(r16[_NHALF:], 16)
    d_ref[...] = w.reshape(_NHALF * 8, 128)


def _sc_triplet_body(dpacked, iap_hbm, ian_hbm, sha_hbm, out_hbm,
                     iap, ian, sha, wap, wan, accv, sem1, sem2):
    wid = lax.axis_index("s") * _NC + lax.axis_index("c")
    base = wid * _BPW
    cp1 = pltpu.async_copy(iap_hbm.at[pl.ds(base, _BPW)], iap, sem1)
    cp2 = pltpu.async_copy(ian_hbm.at[pl.ds(base, _BPW)], ian, sem2)
    cp3 = pltpu.async_copy(sha_hbm.at[pl.ds(base, _BPW)], sha, sem1)
    cp1.wait()
    cp2.wait()
    copies = []
    for c in range(_NCH):
        sl = pl.ds(c * _CH, _CH)
        copies.append(pltpu.async_copy(dpacked.at[iap.at[sl]], wap.at[sl], sem1))
        copies.append(pltpu.async_copy(dpacked.at[ian.at[sl]], wan.at[sl], sem2))
    cp3.wait()

    mask = jnp.full((_LANES,), jnp.int32(-65536))

    def unpack(wv, sh):
        return lax.bitcast_convert_type(lax.shift_left(wv, sh) & mask,
                                        jnp.float32)

    def step(v, acc):
        sl = pl.ds(v * _LANES, _LANES)
        sh = sha[sl]
        d_ap = _sqrt16(jnp.maximum(unpack(wap[sl], sh), 0.0))
        d_an = _sqrt16(jnp.maximum(unpack(wan[sl], sh), 0.0))
        return acc + jnp.maximum(d_ap - d_an + _MARGIN, 0.0)

    for cp in copies:
        cp.wait()
    accv[...] = lax.fori_loop(0, _BPW // _LANES,
                              step, jnp.zeros((_LANES,), jnp.float32))
    pltpu.sync_copy(accv, out_hbm.at[wid])


_sc_triplet = functools.partial(
    pl.kernel,
    out_type=jax.ShapeDtypeStruct((_NW, _LANES), jnp.float32),
    mesh=plsc.VectorSubcoreMesh(core_axis_name="c", subcore_axis_name="s",
                                num_cores=_NC),
    scratch_types=[
        pltpu.VMEM((_BPW,), jnp.int32),
        pltpu.VMEM((_BPW,), jnp.int32),
        pltpu.VMEM((_BPW,), jnp.int32),
        pltpu.VMEM((_BPW,), jnp.int32),
        pltpu.VMEM((_BPW,), jnp.int32),
        pltpu.VMEM((_LANES,), jnp.float32),
        pltpu.SemaphoreType.DMA,
        pltpu.SemaphoreType.DMA,
    ],
)(_sc_triplet_body)


def kernel(a, p, n, emb_table):
    dist, iap, ian, sha = pl.pallas_call(
        _dist_matrix_body,
        out_shape=(
            jax.ShapeDtypeStruct((_NHALF * 8, 128), jnp.int32),
            jax.ShapeDtypeStruct((_BATCH,), jnp.int32),
            jax.ShapeDtypeStruct((_BATCH,), jnp.int32),
            jax.ShapeDtypeStruct((_BATCH,), jnp.int32),
        ),
    )(emb_table, a, p, n)
    partials = _sc_triplet(dist.reshape(_NWORDS), iap, ian, sha)
    return jnp.sum(partials) / _BATCH
